# Initial kernel scaffold; baseline (speedup 1.0000x reference)
import jax
import jax.numpy as jnp
from jax.experimental import pallas as pl


def kernel(ego_embedding, edge_index, norm, W1, W2):
    raise NotImplementedError("write your pallas kernel here")


# speedup vs baseline: 15.9769x; 15.9769x over previous
import functools

import jax
import jax.numpy as jnp
from jax import lax
from jax.experimental import pallas as pl
from jax.experimental.pallas import tpu as pltpu
from jax.experimental.pallas import tpu_sc as plsc

NC = 2
NS = 16
LANES = 16
CHUNK = 128


def _scale_kernel(ego_ref, norm_ref, g_ref):
    g_ref[...] = ego_ref[...] * norm_ref[...]


def _scale(ego_p, norm_p, block=512):
    npad, d = ego_p.shape
    grid = npad // block
    return pl.pallas_call(
        _scale_kernel,
        grid=(grid,),
        in_specs=[
            pl.BlockSpec((block, d), lambda i: (i, 0)),
            pl.BlockSpec((block, 1), lambda i: (i, 0)),
        ],
        out_specs=pl.BlockSpec((block, d), lambda i: (i, 0)),
        out_shape=jax.ShapeDtypeStruct((npad, d), jnp.float32),
    )(ego_p, norm_p)


def _make_segsum(npad, d, per_w):
    steps = per_w // CHUNK
    rows_per_tile = npad // NS
    zcopies = rows_per_tile // CHUNK
    mesh = plsc.VectorSubcoreMesh(core_axis_name="c", subcore_axis_name="s")

    @functools.partial(
        pl.kernel,
        out_type=jax.ShapeDtypeStruct((NC, npad, d), jnp.float32),
        mesh=mesh,
        scratch_types=[
            pltpu.VMEM((CHUNK,), jnp.int32),
            pltpu.VMEM((CHUNK,), jnp.int32),
            pltpu.VMEM((CHUNK, d), jnp.float32),
            pltpu.VMEM_SHARED((npad, d), jnp.float32),
            pltpu.SemaphoreType.DMA,
        ],
    )
    def segsum(g_hbm, src_hbm, dst_hbm, out_hbm, idx_v, dst_v, rows_v, acc_sh, sem):
        c = lax.axis_index("c")
        s = lax.axis_index("s")

        def zrow(i, _):
            def zcol(j, _):
                rows_v[i, pl.ds(j * LANES, LANES)] = jnp.zeros((LANES,), jnp.float32)
                return 0
            return lax.fori_loop(0, d // LANES, zcol, 0)
        lax.fori_loop(0, CHUNK, zrow, 0)

        zbase = s * rows_per_tile
        for k in range(zcopies):
            pltpu.sync_copy(rows_v, acc_sh.at[pl.ds(zbase + k * CHUNK, CHUNK)])
        plsc.subcore_barrier()

        wid = s * NC + c
        ebase = wid * per_w

        def step(t, _):
            off = ebase + t * CHUNK
            pltpu.sync_copy(src_hbm.at[pl.ds(off, CHUNK)], idx_v)
            pltpu.sync_copy(dst_hbm.at[pl.ds(off, CHUNK)], dst_v)
            pltpu.async_copy(g_hbm.at[idx_v], rows_v, sem).wait()
            pltpu.sync_copy(rows_v, acc_sh.at[dst_v], add=True)
            return 0
        lax.fori_loop(0, steps, step, 0)

        plsc.subcore_barrier()
        pltpu.sync_copy(
            acc_sh.at[pl.ds(zbase, rows_per_tile)],
            out_hbm.at[c, pl.ds(zbase, rows_per_tile)],
        )

    return segsum


def _finish_kernel(sa_ref, sb_ref, ego_ref, norm_ref, w1_ref, w2_ref, out_ref):
    s = sa_ref[...] + sb_ref[...]
    ego = ego_ref[...]
    nrm = norm_ref[...]
    t1 = ego + nrm * s
    t2 = (nrm * ego) * s
    h = jnp.dot(t1, w1_ref[...], preferred_element_type=jnp.float32)
    h += jnp.dot(t2, w2_ref[...], preferred_element_type=jnp.float32)
    h = jnp.where(h >= 0, h, 0.2 * h)
    denom = jnp.sqrt(jnp.sum(h * h, axis=1, keepdims=True))
    out_ref[...] = h / jnp.maximum(denom, 1e-12)


def _finish(sa, sb, ego_p, norm_p, w1, w2, n, block=400):
    d = ego_p.shape[1]
    grid = (n + block - 1) // block
    return pl.pallas_call(
        _finish_kernel,
        grid=(grid,),
        in_specs=[
            pl.BlockSpec((block, d), lambda i: (i, 0)),
            pl.BlockSpec((block, d), lambda i: (i, 0)),
            pl.BlockSpec((block, d), lambda i: (i, 0)),
            pl.BlockSpec((block, 1), lambda i: (i, 0)),
            pl.BlockSpec((d, d), lambda i: (0, 0)),
            pl.BlockSpec((d, d), lambda i: (0, 0)),
        ],
        out_specs=pl.BlockSpec((block, d), lambda i: (i, 0)),
        out_shape=jax.ShapeDtypeStruct((n, d), jnp.float32),
    )(sa, sb, ego_p, norm_p, w1, w2)


@jax.jit
def kernel(ego_embedding, edge_index, norm, W1, W2):
    n, d = ego_embedding.shape
    e = edge_index.shape[1]

    npad = -(-n // (NS * CHUNK)) * (NS * CHUNK)
    per_w = -(-e // (NC * NS * CHUNK)) * CHUNK
    epad = per_w * NC * NS

    src = edge_index[0].astype(jnp.int32)
    dst = edge_index[1].astype(jnp.int32)
    src = jnp.pad(src, (0, epad - e), constant_values=n)
    dst = jnp.pad(dst, (0, epad - e), constant_values=0)

    ego_p = jnp.pad(ego_embedding, ((0, npad - n), (0, 0)))
    norm_p = jnp.pad(norm, ((0, npad - n), (0, 0)))

    g = _scale(ego_p, norm_p)
    parts = _make_segsum(npad, d, per_w)(g, src, dst)
    return _finish(parts[0], parts[1], ego_p, norm_p, W1, W2, n)


# speedup vs baseline: 16.7826x; 1.0504x over previous
import functools

import jax
import jax.numpy as jnp
from jax import lax
from jax.experimental import pallas as pl
from jax.experimental.pallas import tpu as pltpu
from jax.experimental.pallas import tpu_sc as plsc

NC = 2
NS = 16
LANES = 16
CHUNK = 128
PHASES = 2


def _scale_kernel(ego_ref, norm_ref, g_ref):
    g_ref[...] = ego_ref[...] * norm_ref[...]


def _scale(ego_p, norm_p):
    npad, d = ego_p.shape
    block = npad // 16
    grid = 16
    return pl.pallas_call(
        _scale_kernel,
        grid=(grid,),
        in_specs=[
            pl.BlockSpec((block, d), lambda i: (i, 0)),
            pl.BlockSpec((block, 1), lambda i: (i, 0)),
        ],
        out_specs=pl.BlockSpec((block, d), lambda i: (i, 0)),
        out_shape=jax.ShapeDtypeStruct((npad, d), jnp.float32),
    )(ego_p, norm_p)


def _make_segsum(npad, d, steps):
    rows_per_tile = npad // NS
    zcopies = rows_per_tile // CHUNK
    zrem = rows_per_tile % CHUNK
    hsteps = steps // PHASES
    mesh = plsc.VectorSubcoreMesh(core_axis_name="c", subcore_axis_name="s")

    @functools.partial(
        pl.kernel,
        out_type=jax.ShapeDtypeStruct((NC, npad, d), jnp.float32),
        mesh=mesh,
        scratch_types=[
            pltpu.VMEM((hsteps, CHUNK), jnp.int32),
            pltpu.VMEM((hsteps, CHUNK), jnp.int32),
            pltpu.VMEM((2, CHUNK, d), jnp.float32),
            pltpu.VMEM_SHARED((npad, d), jnp.float32),
            pltpu.SemaphoreType.DMA,
            pltpu.SemaphoreType.DMA,
        ],
    )
    def segsum(g_hbm, src_hbm, dst_hbm, out_hbm, src_all, dst_all, rows, acc_sh,
               sem0, sem1):
        c = lax.axis_index("c")
        s = lax.axis_index("s")
        sems = (sem0, sem1)

        def zrow(i, _):
            def zcol(j, _):
                rows[0, i, pl.ds(j * LANES, LANES)] = jnp.zeros((LANES,), jnp.float32)
                return 0
            return lax.fori_loop(0, d // LANES, zcol, 0)
        lax.fori_loop(0, CHUNK, zrow, 0)

        zbase = s * rows_per_tile
        for k in range(zcopies):
            pltpu.sync_copy(rows.at[0], acc_sh.at[pl.ds(zbase + k * CHUNK, CHUNK)])
        if zrem:
            pltpu.sync_copy(
                rows.at[0, pl.ds(0, zrem)],
                acc_sh.at[pl.ds(zbase + zcopies * CHUNK, zrem)],
            )

        wid = s * NC + c
        plsc.subcore_barrier()

        def gather(t, b):
            pltpu.async_copy(g_hbm.at[src_all.at[t]], rows.at[b], sems[b])

        def wait_gather(t, b):
            pltpu.make_async_copy(g_hbm.at[src_all.at[t]], rows.at[b], sems[b]).wait()

        def scat(t, b):
            pltpu.sync_copy(rows.at[b], acc_sh.at[dst_all.at[t]], add=True)

        def pair(p, _):
            for b in range(2):
                t = 2 * p + b
                wait_gather(t, b)
                scat(t, b)
                gather(t + 2, b)
            return 0

        for ph in range(PHASES):
            pltpu.sync_copy(src_hbm.at[wid, pl.ds(ph * hsteps, hsteps)], src_all)
            pltpu.sync_copy(dst_hbm.at[wid, pl.ds(ph * hsteps, hsteps)], dst_all)
            gather(0, 0)
            gather(1, 1)
            lax.fori_loop(0, hsteps // 2 - 1, pair, 0)
            for b in range(2):
                t = hsteps - 2 + b
                wait_gather(t, b)
                scat(t, b)

        plsc.subcore_barrier()
        pltpu.sync_copy(
            acc_sh.at[pl.ds(zbase, rows_per_tile)],
            out_hbm.at[c, pl.ds(zbase, rows_per_tile)],
        )

    return segsum


def _finish_kernel(sa_ref, sb_ref, ego_ref, norm_ref, w1_ref, w2_ref, out_ref):
    s = sa_ref[...] + sb_ref[...]
    ego = ego_ref[...]
    nrm = norm_ref[...]
    t1 = ego + nrm * s
    t2 = (nrm * ego) * s
    h = jnp.dot(t1, w1_ref[...], preferred_element_type=jnp.float32)
    h += jnp.dot(t2, w2_ref[...], preferred_element_type=jnp.float32)
    h = jnp.where(h >= 0, h, 0.2 * h)
    denom = jnp.sqrt(jnp.sum(h * h, axis=1, keepdims=True))
    out_ref[...] = h / jnp.maximum(denom, 1e-12)


def _finish(sa, sb, ego_p, norm_p, w1, w2, n, block=400):
    d = ego_p.shape[1]
    grid = (n + block - 1) // block
    return pl.pallas_call(
        _finish_kernel,
        grid=(grid,),
        in_specs=[
            pl.BlockSpec((block, d), lambda i: (i, 0)),
            pl.BlockSpec((block, d), lambda i: (i, 0)),
            pl.BlockSpec((block, d), lambda i: (i, 0)),
            pl.BlockSpec((block, 1), lambda i: (i, 0)),
            pl.BlockSpec((d, d), lambda i: (0, 0)),
            pl.BlockSpec((d, d), lambda i: (0, 0)),
        ],
        out_specs=pl.BlockSpec((block, d), lambda i: (i, 0)),
        out_shape=jax.ShapeDtypeStruct((n, d), jnp.float32),
    )(sa, sb, ego_p, norm_p, w1, w2)


@jax.jit
def kernel(ego_embedding, edge_index, norm, W1, W2):
    n, d = ego_embedding.shape
    e = edge_index.shape[1]

    npad = -(-n // (NS * 8)) * (NS * 8)
    nw = NC * NS
    per_w = -(-e // (nw * PHASES * 2 * CHUNK)) * (PHASES * 2 * CHUNK)
    steps = per_w // CHUNK
    epad = per_w * nw

    src = edge_index[0].astype(jnp.int32)
    dst = edge_index[1].astype(jnp.int32)
    src = jnp.pad(src, (0, epad - e), constant_values=n).reshape(nw, steps, CHUNK)
    dst = jnp.pad(dst, (0, epad - e), constant_values=0).reshape(nw, steps, CHUNK)

    ego_p = jnp.pad(ego_embedding, ((0, npad - n), (0, 0)))
    norm_p = jnp.pad(norm, ((0, npad - n), (0, 0)))

    g = _scale(ego_p, norm_p)
    parts = _make_segsum(npad, d, steps)(g, src, dst)
    return _finish(parts[0], parts[1], ego_p, norm_p, W1, W2, n)
